# Initial kernel scaffold; baseline (speedup 1.0000x reference)
import jax
import jax.numpy as jnp
from jax.experimental import pallas as pl


def kernel(x, edge_index, W1, as1, ad1, b1, Wl1, bl1, W2, as2, ad2, b2, Wl2, bl2, W3, as3, ad3, b3, Wl3, bl3):
    raise NotImplementedError("write your pallas kernel here")


# speedup vs baseline: 11.7542x; 11.7542x over previous
import functools

import jax
import jax.numpy as jnp
from jax import lax
from jax.experimental import pallas as pl
from jax.experimental.pallas import tpu as pltpu
from jax.experimental.pallas import tpu_sc as plsc

_N = 10000
_NP = 10240
_E_TRUE = 170000
_NTILES = 32
_EPT = 5344
_NB = _EPT // 16
_E_PAD = _NTILES * _EPT
_BN = 1000


def _mm_body(x_ref, w_ref, s_ref, d_ref, h_ref, as_ref, ad_ref):
    c = pl.program_id(1)
    h = jnp.dot(x_ref[...], w_ref[...], preferred_element_type=jnp.float32)
    h_ref[0] = h
    sv = jnp.dot(h, s_ref[...], preferred_element_type=jnp.float32)
    dv = jnp.dot(h, d_ref[...], preferred_element_type=jnp.float32)

    @pl.when(c == 0)
    def _():
        as_ref[...] = sv
        ad_ref[...] = dv

    @pl.when(c > 0)
    def _():
        as_ref[...] = as_ref[...] + sv
        ad_ref[...] = ad_ref[...] + dv


def _mm(x, w, att_s, att_d, c_tot, H):
    k = x.shape[1]
    return pl.pallas_call(
        _mm_body,
        grid=(_N // _BN, c_tot),
        in_specs=[
            pl.BlockSpec((_BN, k), lambda n, c: (n, 0)),
            pl.BlockSpec((k, 128), lambda n, c: (0, c)),
            pl.BlockSpec((128, H), lambda n, c: (c, 0)),
            pl.BlockSpec((128, H), lambda n, c: (c, 0)),
        ],
        out_specs=[
            pl.BlockSpec((1, _BN, 128), lambda n, c: (c, n, 0)),
            pl.BlockSpec((_BN, H), lambda n, c: (n, 0)),
            pl.BlockSpec((_BN, H), lambda n, c: (n, 0)),
        ],
        out_shape=[
            jax.ShapeDtypeStruct((c_tot, _N, 128), jnp.float32),
            jax.ShapeDtypeStruct((_N, H), jnp.float32),
            jax.ShapeDtypeStruct((_N, H), jnp.float32),
        ],
    )(x, w, att_s, att_d)


def _make_edge_a(H):
    mesh = plsc.VectorSubcoreMesh(core_axis_name="c", subcore_axis_name="s")

    @functools.partial(
        pl.kernel,
        out_type=jax.ShapeDtypeStruct((_NTILES * H * _NP,), jnp.float32),
        mesh=mesh,
        compiler_params=pltpu.CompilerParams(needs_layout_passes=False),
        scratch_types=[
            pltpu.VMEM((_EPT,), jnp.int32),
            pltpu.VMEM((_EPT,), jnp.int32),
            pltpu.VMEM((_N,), jnp.float32),
            pltpu.VMEM((_N,), jnp.float32),
            pltpu.VMEM((_NP,), jnp.float32),
        ],
    )
    def edge_a(asrc, adst, src, dst, den_out,
               src_v, dst_v, as_v, ad_v, den_v):
        core = lax.axis_index("c")
        sub = lax.axis_index("s")
        wid = core * 16 + sub
        base = pl.multiple_of(wid * _EPT, 8)
        pltpu.sync_copy(src.at[pl.ds(base, _EPT)], src_v)
        pltpu.sync_copy(dst.at[pl.ds(base, _EPT)], dst_v)

        for h in range(H):
            pltpu.sync_copy(asrc.at[pl.ds(h * _N, _N)], as_v)
            pltpu.sync_copy(adst.at[pl.ds(h * _N, _N)], ad_v)

            def _zl(i, _):
                den_v[pl.ds(i * 16, 16)] = jnp.zeros((16,), jnp.float32)
                return 0

            lax.fori_loop(0, _NP // 16, _zl, 0)

            def _el(b, _):
                sl16 = pl.ds(b * 16, 16)
                s16 = src_v[sl16]
                d16 = dst_v[sl16]
                a = plsc.load_gather(as_v, [s16]) + plsc.load_gather(ad_v, [d16])
                a = jnp.where(a >= 0, a, a * jnp.float32(0.2))
                v = jnp.exp(a)
                gi = base + b * 16 + lax.iota(jnp.int32, 16)
                v = jnp.where(gi < _E_TRUE, v, jnp.float32(0.0))
                plsc.addupdate_scatter(den_v, [d16], v)
                return 0

            lax.fori_loop(0, _NB, _el, 0)
            doff = pl.multiple_of((wid * H + h) * _NP, 8)
            pltpu.sync_copy(den_v, den_out.at[pl.ds(doff, _NP)])

    return edge_a


def _inv_body(dp_ref, inv_ref):
    inv_ref[...] = 1.0 / (jnp.sum(dp_ref[...], axis=0) + jnp.float32(1e-16))


def _inv(den_parts, H):
    return pl.pallas_call(
        _inv_body,
        out_shape=jax.ShapeDtypeStruct((H, _NP), jnp.float32),
    )(den_parts)


def _make_edge_b(H, halves, c_feat):
    mesh = plsc.VectorSubcoreMesh(core_axis_name="c", subcore_axis_name="s")

    @functools.partial(
        pl.kernel,
        out_type=jax.ShapeDtypeStruct((2, c_feat, _NP, 128), jnp.float32),
        mesh=mesh,
        compiler_params=pltpu.CompilerParams(needs_layout_passes=False),
        scratch_types=[
            pltpu.VMEM((_EPT,), jnp.int32),
            pltpu.VMEM((_EPT,), jnp.int32),
            pltpu.VMEM((_EPT,), jnp.float32),
            pltpu.VMEM((_N,), jnp.float32),
            pltpu.VMEM((_N,), jnp.float32),
            pltpu.VMEM((2, 16, 128), jnp.float32),
            pltpu.VMEM((2, 16), jnp.int32),
            pltpu.VMEM((16,), jnp.int32),
            pltpu.VMEM((640,), jnp.float32),
            pltpu.VMEM_SHARED((_NP, 128), jnp.float32),
            pltpu.SemaphoreType.DMA,
            pltpu.SemaphoreType.DMA,
        ],
    )
    def edge_b(asrc, adst, src, dst, hflat, inv_hbm, zrow, num_out,
               src_v, dst_v, val_v, as_v, ad_v, rows_v, sidx_v, didx_v,
               invs_v, num_sh, sem0, sem1):
        sems = (sem0, sem1)
        core = lax.axis_index("c")
        sub = lax.axis_index("s")
        wid = core * 16 + sub
        base = pl.multiple_of(wid * _EPT, 8)
        sbase = pl.multiple_of(sub * 640, 8)
        pltpu.sync_copy(src.at[pl.ds(base, _EPT)], src_v)
        pltpu.sync_copy(dst.at[pl.ds(base, _EPT)], dst_v)

        def _chunk(c, _):
            hh = c // halves
            aoff = pl.multiple_of(hh * _N, 8)
            pltpu.sync_copy(asrc.at[pl.ds(aoff, _N)], as_v)
            pltpu.sync_copy(adst.at[pl.ds(aoff, _N)], ad_v)

            def _el(b, _):
                sl16 = pl.ds(b * 16, 16)
                a = (plsc.load_gather(as_v, [src_v[sl16]])
                     + plsc.load_gather(ad_v, [dst_v[sl16]]))
                a = jnp.where(a >= 0, a, a * jnp.float32(0.2))
                v = jnp.exp(a)
                gi = base + b * 16 + lax.iota(jnp.int32, 16)
                v = jnp.where(gi < _E_TRUE, v, jnp.float32(0.0))
                val_v[sl16] = v
                return 0

            lax.fori_loop(0, _NB, _el, 0)
            def _zz(r, _):
                pltpu.sync_copy(zrow, num_sh.at[pl.ds(sbase + r * 128, 128)])
                return 0

            lax.fori_loop(0, 5, _zz, 0)
            plsc.subcore_barrier()
            for slot in range(2):
                sidx_v[slot] = src_v[pl.ds(slot * 16, 16)] + c * _N
                pltpu.async_copy(hflat.at[sidx_v.at[slot]], rows_v.at[slot],
                                 sems[slot])

            def _pb(i, _):
                for slot in range(2):
                    b = i * 2 + slot
                    pltpu.make_async_copy(
                        hflat.at[sidx_v.at[slot]], rows_v.at[slot],
                        sems[slot]).wait()
                    didx_v[...] = dst_v[pl.ds(b * 16, 16)]
                    for j in range(16):
                        sp = plsc.load_gather(
                            val_v, [jnp.zeros((16,), jnp.int32) + (b * 16 + j)])
                        for g in range(8):
                            sl16 = pl.ds(g * 16, 16)
                            rows_v[slot, j, sl16] = rows_v[slot, j, sl16] * sp
                    pltpu.sync_copy(rows_v.at[slot], num_sh.at[didx_v], add=True)
                    bnext = jnp.minimum(b + 2, _NB - 1)
                    sidx_v[slot] = src_v[pl.ds(bnext * 16, 16)] + c * _N
                    pltpu.async_copy(hflat.at[sidx_v.at[slot]], rows_v.at[slot],
                                     sems[slot])
                return 0

            lax.fori_loop(0, _NB // 2, _pb, 0)
            for slot in range(2):
                pltpu.make_async_copy(
                    hflat.at[sidx_v.at[slot]], rows_v.at[slot], sems[slot]).wait()
            plsc.subcore_barrier()
            ioff = pl.multiple_of(hh * _NP + sbase, 8)
            pltpu.sync_copy(inv_hbm.at[pl.ds(ioff, 640)], invs_v)

            def _wr(r, _):
                wb = rows_v.at[0]
                pltpu.sync_copy(num_sh.at[pl.ds(sbase + r * 16, 16)], wb)
                for j in range(16):
                    sp = plsc.load_gather(
                        invs_v, [jnp.zeros((16,), jnp.int32) + (r * 16 + j)])
                    for g in range(8):
                        sl16 = pl.ds(g * 16, 16)
                        rows_v[0, j, sl16] = rows_v[0, j, sl16] * sp
                pltpu.sync_copy(wb, num_out.at[core, c, pl.ds(sbase + r * 16, 16)])
                return 0

            lax.fori_loop(0, 40, _wr, 0)
            return 0

        lax.fori_loop(0, c_feat, _chunk, 0)

    return edge_b


_edge_a4 = _make_edge_a(H=4)
_edge_a6 = _make_edge_a(H=6)
_edge_b12 = _make_edge_b(H=4, halves=2, c_feat=8)
_edge_b3 = _make_edge_b(H=6, halves=1, c_feat=6)


def _c12_body(num_ref, hc_ref, bg_ref, bl_ref, out_ref):
    c = pl.program_id(0)
    v = num_ref[0, 0] + num_ref[1, 0]
    v = v + bg_ref[c][None, :] + hc_ref[0] + bl_ref[c][None, :]
    out_ref[...] = jnp.where(v > 0, v, jnp.exp(jnp.minimum(v, 0.0)) - 1.0)


def _c12(num, hc, bg, bl, c_feat):
    return pl.pallas_call(
        _c12_body,
        grid=(c_feat, _N // _BN),
        in_specs=[
            pl.BlockSpec((2, 1, _BN, 128), lambda c, n: (0, c, n, 0)),
            pl.BlockSpec((1, _BN, 128), lambda c, n: (c_feat + c, n, 0)),
            pl.BlockSpec((c_feat, 128), lambda c, n: (0, 0)),
            pl.BlockSpec((c_feat, 128), lambda c, n: (0, 0)),
        ],
        out_specs=pl.BlockSpec((_BN, 128), lambda c, n: (n, c)),
        out_shape=jax.ShapeDtypeStruct((_N, c_feat * 128), jnp.float32),
    )(num, hc, bg, bl)


def _c3_body(num_ref, hc_ref, b3_ref, bl3_ref, out_ref):
    acc = num_ref[0, 0] + num_ref[1, 0]
    for h in range(1, 6):
        acc = acc + num_ref[0, h] + num_ref[1, h]
    out_ref[...] = (acc * jnp.float32(1.0 / 6.0)
                    + b3_ref[0][None, :] + hc_ref[0] + bl3_ref[0][None, :])


def _c3(num, hc, b3, bl3):
    return pl.pallas_call(
        _c3_body,
        grid=(_N // _BN,),
        in_specs=[
            pl.BlockSpec((2, 6, _BN, 128), lambda n: (0, 0, n, 0)),
            pl.BlockSpec((1, _BN, 128), lambda n: (6, n, 0)),
            pl.BlockSpec((1, 128), lambda n: (0, 0)),
            pl.BlockSpec((1, 128), lambda n: (0, 0)),
        ],
        out_specs=pl.BlockSpec((_BN, 128), lambda n: (n, 0)),
        out_shape=jax.ShapeDtypeStruct((_N, 128), jnp.float32),
    )(num, hc, b3, bl3)


def _att_mats(att, c_tot, H, halves):
    rows = att.reshape(H * halves, 128)
    rows = jnp.concatenate(
        [rows, jnp.zeros((c_tot - H * halves, 128), jnp.float32)], axis=0)
    eye = jnp.repeat(jnp.eye(H, dtype=jnp.float32), halves, axis=0)
    oh = jnp.concatenate([eye, jnp.zeros((c_tot - H * halves, H))], axis=0)
    return (rows[:, :, None] * oh[:, None, :]).reshape(c_tot * 128, H)


def _gat_layer(x, W, att_s, att_d, Wl, edge_a, edge_b, src, dst, zrow,
               c_tot, c_feat, H, halves):
    wc = jnp.concatenate([W, Wl], axis=1)
    a_s = _att_mats(att_s, c_tot, H, halves)
    a_d = _att_mats(att_d, c_tot, H, halves)
    hc, s, d = _mm(x, wc, a_s, a_d, c_tot, H)
    st, dt = s.T.reshape(-1), d.T.reshape(-1)
    den = edge_a(st, dt, src, dst)
    inv = _inv(den.reshape(_NTILES, H, _NP), H)
    num = edge_b(st, dt, src, dst, hc.reshape(c_tot * _N, 128),
                 inv.reshape(-1), zrow)
    return hc, num


def kernel(x, edge_index, W1, as1, ad1, b1, Wl1, bl1, W2, as2, ad2, b2,
           Wl2, bl2, W3, as3, ad3, b3, Wl3, bl3):
    ar = jnp.arange(_N, dtype=jnp.int32)
    src = jnp.concatenate([edge_index[0].astype(jnp.int32), ar])
    dst = jnp.concatenate([edge_index[1].astype(jnp.int32), ar])
    pad = _E_PAD - _E_TRUE
    src = jnp.pad(src, (0, pad))
    dst = jnp.pad(dst, (0, pad))
    zrow = jnp.zeros((128, 128), jnp.float32)

    hc, num = _gat_layer(x, W1, as1, ad1, Wl1, _edge_a4, _edge_b12,
                         src, dst, zrow, c_tot=16, c_feat=8, H=4, halves=2)
    h1 = _c12(num, hc, b1.reshape(8, 128), bl1.reshape(8, 128), c_feat=8)

    hc, num = _gat_layer(h1, W2, as2, ad2, Wl2, _edge_a4, _edge_b12,
                         src, dst, zrow, c_tot=16, c_feat=8, H=4, halves=2)
    h2 = _c12(num, hc, b2.reshape(8, 128), bl2.reshape(8, 128), c_feat=8)

    hc, num = _gat_layer(h2, W3, as3, ad3, Wl3, _edge_a6, _edge_b3,
                         src, dst, zrow, c_tot=7, c_feat=6, H=6, halves=1)
    return _c3(num, hc, b3.reshape(1, 128), bl3.reshape(1, 128))


# speedup vs baseline: 12.6233x; 1.0739x over previous
import functools

import jax
import jax.numpy as jnp
from jax import lax
from jax.experimental import pallas as pl
from jax.experimental.pallas import tpu as pltpu
from jax.experimental.pallas import tpu_sc as plsc

_N = 10000
_NP = 10240
_E_TRUE = 170000
_NTILES = 32
_EPT = 5344
_NB = _EPT // 16
_E_PAD = _NTILES * _EPT
_BN = 1000


def _mm_body(x_ref, w_ref, s_ref, d_ref, h_ref, as_ref, ad_ref):
    c = pl.program_id(1)
    h = jnp.dot(x_ref[...], w_ref[...], preferred_element_type=jnp.float32)
    h_ref[0] = h
    sv = jnp.dot(h, s_ref[...], preferred_element_type=jnp.float32)
    dv = jnp.dot(h, d_ref[...], preferred_element_type=jnp.float32)

    @pl.when(c == 0)
    def _():
        as_ref[...] = sv
        ad_ref[...] = dv

    @pl.when(c > 0)
    def _():
        as_ref[...] = as_ref[...] + sv
        ad_ref[...] = ad_ref[...] + dv


def _mm(x, w, att_s, att_d, c_tot, H):
    k = x.shape[1]
    return pl.pallas_call(
        _mm_body,
        grid=(_N // _BN, c_tot),
        in_specs=[
            pl.BlockSpec((_BN, k), lambda n, c: (n, 0)),
            pl.BlockSpec((k, 128), lambda n, c: (0, c)),
            pl.BlockSpec((128, H), lambda n, c: (c, 0)),
            pl.BlockSpec((128, H), lambda n, c: (c, 0)),
        ],
        out_specs=[
            pl.BlockSpec((1, _BN, 128), lambda n, c: (c, n, 0)),
            pl.BlockSpec((_BN, H), lambda n, c: (n, 0)),
            pl.BlockSpec((_BN, H), lambda n, c: (n, 0)),
        ],
        out_shape=[
            jax.ShapeDtypeStruct((c_tot, _N, 128), jnp.float32),
            jax.ShapeDtypeStruct((_N, H), jnp.float32),
            jax.ShapeDtypeStruct((_N, H), jnp.float32),
        ],
    )(x, w, att_s, att_d)


def _make_edge_a(H):
    mesh = plsc.VectorSubcoreMesh(core_axis_name="c", subcore_axis_name="s")

    @functools.partial(
        pl.kernel,
        out_type=jax.ShapeDtypeStruct((_NTILES * H * _NP,), jnp.float32),
        mesh=mesh,
        compiler_params=pltpu.CompilerParams(needs_layout_passes=False),
        scratch_types=[
            pltpu.VMEM((_EPT,), jnp.int32),
            pltpu.VMEM((_EPT,), jnp.int32),
            pltpu.VMEM((_N,), jnp.float32),
            pltpu.VMEM((_N,), jnp.float32),
            pltpu.VMEM((_NP,), jnp.float32),
        ],
    )
    def edge_a(asrc, adst, src, dst, den_out,
               src_v, dst_v, as_v, ad_v, den_v):
        core = lax.axis_index("c")
        sub = lax.axis_index("s")
        wid = core * 16 + sub
        base = pl.multiple_of(wid * _EPT, 8)
        pltpu.sync_copy(src.at[pl.ds(base, _EPT)], src_v)
        pltpu.sync_copy(dst.at[pl.ds(base, _EPT)], dst_v)

        for h in range(H):
            pltpu.sync_copy(asrc.at[pl.ds(h * _N, _N)], as_v)
            pltpu.sync_copy(adst.at[pl.ds(h * _N, _N)], ad_v)

            def _zl(i, _):
                den_v[pl.ds(i * 16, 16)] = jnp.zeros((16,), jnp.float32)
                return 0

            lax.fori_loop(0, _NP // 16, _zl, 0)

            def _el(b, _):
                sl16 = pl.ds(b * 16, 16)
                s16 = src_v[sl16]
                d16 = dst_v[sl16]
                a = plsc.load_gather(as_v, [s16]) + plsc.load_gather(ad_v, [d16])
                a = jnp.where(a >= 0, a, a * jnp.float32(0.2))
                v = jnp.exp(a)
                gi = base + b * 16 + lax.iota(jnp.int32, 16)
                v = jnp.where(gi < _E_TRUE, v, jnp.float32(0.0))
                plsc.addupdate_scatter(den_v, [d16], v)
                return 0

            lax.fori_loop(0, _NB, _el, 0)
            doff = pl.multiple_of((wid * H + h) * _NP, 8)
            pltpu.sync_copy(den_v, den_out.at[pl.ds(doff, _NP)])

    return edge_a


def _inv_body(dp_ref, inv_ref):
    inv_ref[...] = 1.0 / (jnp.sum(dp_ref[...], axis=0) + jnp.float32(1e-16))


def _inv(den_parts, H):
    return pl.pallas_call(
        _inv_body,
        out_shape=jax.ShapeDtypeStruct((H, _NP), jnp.float32),
    )(den_parts)


def _make_edge_b(H, halves, c_feat):
    mesh = plsc.VectorSubcoreMesh(core_axis_name="c", subcore_axis_name="s")

    @functools.partial(
        pl.kernel,
        out_type=jax.ShapeDtypeStruct((2, c_feat, _NP, 128), jnp.float32),
        mesh=mesh,
        compiler_params=pltpu.CompilerParams(needs_layout_passes=False),
        scratch_types=[
            pltpu.VMEM((_EPT,), jnp.int32),
            pltpu.VMEM((_EPT,), jnp.int32),
            pltpu.VMEM((_EPT,), jnp.float32),
            pltpu.VMEM((_N,), jnp.float32),
            pltpu.VMEM((_N,), jnp.float32),
            pltpu.VMEM((2, 16, 128), jnp.float32),
            pltpu.VMEM((2, 16, 128), jnp.float32),
            pltpu.VMEM((2, 16), jnp.int32),
            pltpu.VMEM((2, 16), jnp.int32),
            pltpu.VMEM((640,), jnp.float32),
            pltpu.VMEM_SHARED((_NP, 128), jnp.float32),
            pltpu.SemaphoreType.DMA,
            pltpu.SemaphoreType.DMA,
            pltpu.SemaphoreType.DMA,
            pltpu.SemaphoreType.DMA,
        ],
    )
    def edge_b(asrc, adst, src, dst, hflat, inv_hbm, zrow, num_out,
               src_v, dst_v, val_v, as_v, ad_v, rows_v, srows_v, sidx_v,
               didx_v, invs_v, num_sh, sem0, sem1, sem2, sem3):
        gsems = (sem0, sem1)
        ssems = (sem2, sem3)
        core = lax.axis_index("c")
        sub = lax.axis_index("s")
        wid = core * 16 + sub
        base = pl.multiple_of(wid * _EPT, 8)
        sbase = pl.multiple_of(sub * 640, 8)
        pltpu.sync_copy(src.at[pl.ds(base, _EPT)], src_v)
        pltpu.sync_copy(dst.at[pl.ds(base, _EPT)], dst_v)

        def _chunk(c, _):
            hh = c // halves
            aoff = pl.multiple_of(hh * _N, 8)
            pltpu.sync_copy(asrc.at[pl.ds(aoff, _N)], as_v)
            pltpu.sync_copy(adst.at[pl.ds(aoff, _N)], ad_v)

            def _el(b, _):
                sl16 = pl.ds(b * 16, 16)
                a = (plsc.load_gather(as_v, [src_v[sl16]])
                     + plsc.load_gather(ad_v, [dst_v[sl16]]))
                a = jnp.where(a >= 0, a, a * jnp.float32(0.2))
                v = jnp.exp(a)
                gi = base + b * 16 + lax.iota(jnp.int32, 16)
                v = jnp.where(gi < _E_TRUE, v, jnp.float32(0.0))
                val_v[sl16] = v
                return 0

            lax.fori_loop(0, _NB, _el, 0)
            def _zz(r, _):
                pltpu.sync_copy(zrow, num_sh.at[pl.ds(sbase + r * 128, 128)])
                return 0

            lax.fori_loop(0, 5, _zz, 0)
            plsc.subcore_barrier()
            for slot in range(2):
                sidx_v[slot] = src_v[pl.ds(slot * 16, 16)] + c * _N
                pltpu.async_copy(hflat.at[sidx_v.at[slot]], rows_v.at[slot],
                                 gsems[slot])

            def _pb(i, _):
                for slot in range(2):
                    b = i * 2 + slot
                    pltpu.make_async_copy(
                        hflat.at[sidx_v.at[slot]], rows_v.at[slot],
                        gsems[slot]).wait()

                    @pl.when(i > 0)
                    def _():
                        pltpu.make_async_copy(
                            srows_v.at[slot], num_sh.at[didx_v.at[slot]],
                            ssems[slot]).wait()
                    didx_v[slot] = dst_v[pl.ds(b * 16, 16)]
                    for j in range(16):
                        sp = plsc.load_gather(
                            val_v, [jnp.zeros((16,), jnp.int32) + (b * 16 + j)])
                        for g in range(8):
                            sl16 = pl.ds(g * 16, 16)
                            srows_v[slot, j, sl16] = rows_v[slot, j, sl16] * sp
                    bnext = jnp.minimum(b + 2, _NB - 1)
                    sidx_v[slot] = src_v[pl.ds(bnext * 16, 16)] + c * _N
                    pltpu.async_copy(hflat.at[sidx_v.at[slot]], rows_v.at[slot],
                                     gsems[slot])
                    pltpu.async_copy(srows_v.at[slot], num_sh.at[didx_v.at[slot]],
                                     ssems[slot], add=True)
                return 0

            lax.fori_loop(0, _NB // 2, _pb, 0)
            for slot in range(2):
                pltpu.make_async_copy(
                    hflat.at[sidx_v.at[slot]], rows_v.at[slot], gsems[slot]).wait()
                pltpu.make_async_copy(
                    srows_v.at[slot], num_sh.at[didx_v.at[slot]],
                    ssems[slot]).wait()
            plsc.subcore_barrier()
            ioff = pl.multiple_of(hh * _NP + sbase, 8)
            pltpu.sync_copy(inv_hbm.at[pl.ds(ioff, 640)], invs_v)

            def _wr(r, _):
                wb = rows_v.at[0]
                pltpu.sync_copy(num_sh.at[pl.ds(sbase + r * 16, 16)], wb)
                for j in range(16):
                    sp = plsc.load_gather(
                        invs_v, [jnp.zeros((16,), jnp.int32) + (r * 16 + j)])
                    for g in range(8):
                        sl16 = pl.ds(g * 16, 16)
                        rows_v[0, j, sl16] = rows_v[0, j, sl16] * sp
                pltpu.sync_copy(wb, num_out.at[core, c, pl.ds(sbase + r * 16, 16)])
                return 0

            lax.fori_loop(0, 40, _wr, 0)
            return 0

        lax.fori_loop(0, c_feat, _chunk, 0)

    return edge_b


_edge_a4 = _make_edge_a(H=4)
_edge_a6 = _make_edge_a(H=6)
_edge_b12 = _make_edge_b(H=4, halves=2, c_feat=8)
_edge_b3 = _make_edge_b(H=6, halves=1, c_feat=6)


def _c12_body(num_ref, hc_ref, bg_ref, bl_ref, out_ref):
    c = pl.program_id(0)
    v = num_ref[0, 0] + num_ref[1, 0]
    v = v + bg_ref[c][None, :] + hc_ref[0] + bl_ref[c][None, :]
    out_ref[...] = jnp.where(v > 0, v, jnp.exp(jnp.minimum(v, 0.0)) - 1.0)


def _c12(num, hc, bg, bl, c_feat):
    return pl.pallas_call(
        _c12_body,
        grid=(c_feat, _N // _BN),
        in_specs=[
            pl.BlockSpec((2, 1, _BN, 128), lambda c, n: (0, c, n, 0)),
            pl.BlockSpec((1, _BN, 128), lambda c, n: (c_feat + c, n, 0)),
            pl.BlockSpec((c_feat, 128), lambda c, n: (0, 0)),
            pl.BlockSpec((c_feat, 128), lambda c, n: (0, 0)),
        ],
        out_specs=pl.BlockSpec((_BN, 128), lambda c, n: (n, c)),
        out_shape=jax.ShapeDtypeStruct((_N, c_feat * 128), jnp.float32),
    )(num, hc, bg, bl)


def _c3_body(num_ref, hc_ref, b3_ref, bl3_ref, out_ref):
    acc = num_ref[0, 0] + num_ref[1, 0]
    for h in range(1, 6):
        acc = acc + num_ref[0, h] + num_ref[1, h]
    out_ref[...] = (acc * jnp.float32(1.0 / 6.0)
                    + b3_ref[0][None, :] + hc_ref[0] + bl3_ref[0][None, :])


def _c3(num, hc, b3, bl3):
    return pl.pallas_call(
        _c3_body,
        grid=(_N // _BN,),
        in_specs=[
            pl.BlockSpec((2, 6, _BN, 128), lambda n: (0, 0, n, 0)),
            pl.BlockSpec((1, _BN, 128), lambda n: (6, n, 0)),
            pl.BlockSpec((1, 128), lambda n: (0, 0)),
            pl.BlockSpec((1, 128), lambda n: (0, 0)),
        ],
        out_specs=pl.BlockSpec((_BN, 128), lambda n: (n, 0)),
        out_shape=jax.ShapeDtypeStruct((_N, 128), jnp.float32),
    )(num, hc, b3, bl3)


def _att_mats(att, c_tot, H, halves):
    rows = att.reshape(H * halves, 128)
    rows = jnp.concatenate(
        [rows, jnp.zeros((c_tot - H * halves, 128), jnp.float32)], axis=0)
    eye = jnp.repeat(jnp.eye(H, dtype=jnp.float32), halves, axis=0)
    oh = jnp.concatenate([eye, jnp.zeros((c_tot - H * halves, H))], axis=0)
    return (rows[:, :, None] * oh[:, None, :]).reshape(c_tot * 128, H)


def _gat_layer(x, W, att_s, att_d, Wl, edge_a, edge_b, src, dst, zrow,
               c_tot, c_feat, H, halves):
    wc = jnp.concatenate([W, Wl], axis=1)
    a_s = _att_mats(att_s, c_tot, H, halves)
    a_d = _att_mats(att_d, c_tot, H, halves)
    hc, s, d = _mm(x, wc, a_s, a_d, c_tot, H)
    st, dt = s.T.reshape(-1), d.T.reshape(-1)
    den = edge_a(st, dt, src, dst)
    inv = _inv(den.reshape(_NTILES, H, _NP), H)
    num = edge_b(st, dt, src, dst, hc.reshape(c_tot * _N, 128),
                 inv.reshape(-1), zrow)
    return hc, num


def kernel(x, edge_index, W1, as1, ad1, b1, Wl1, bl1, W2, as2, ad2, b2,
           Wl2, bl2, W3, as3, ad3, b3, Wl3, bl3):
    ar = jnp.arange(_N, dtype=jnp.int32)
    src = jnp.concatenate([edge_index[0].astype(jnp.int32), ar])
    dst = jnp.concatenate([edge_index[1].astype(jnp.int32), ar])
    pad = _E_PAD - _E_TRUE
    src = jnp.pad(src, (0, pad))
    dst = jnp.pad(dst, (0, pad))
    zrow = jnp.zeros((128, 128), jnp.float32)

    hc, num = _gat_layer(x, W1, as1, ad1, Wl1, _edge_a4, _edge_b12,
                         src, dst, zrow, c_tot=16, c_feat=8, H=4, halves=2)
    h1 = _c12(num, hc, b1.reshape(8, 128), bl1.reshape(8, 128), c_feat=8)

    hc, num = _gat_layer(h1, W2, as2, ad2, Wl2, _edge_a4, _edge_b12,
                         src, dst, zrow, c_tot=16, c_feat=8, H=4, halves=2)
    h2 = _c12(num, hc, b2.reshape(8, 128), bl2.reshape(8, 128), c_feat=8)

    hc, num = _gat_layer(h2, W3, as3, ad3, Wl3, _edge_a6, _edge_b3,
                         src, dst, zrow, c_tot=7, c_feat=6, H=6, halves=1)
    return _c3(num, hc, b3.reshape(1, 128), bl3.reshape(1, 128))


# speedup vs baseline: 13.1780x; 1.0439x over previous
import functools

import jax
import jax.numpy as jnp
from jax import lax
from jax.experimental import pallas as pl
from jax.experimental.pallas import tpu as pltpu
from jax.experimental.pallas import tpu_sc as plsc

_N = 10000
_NP = 10240
_E_TRUE = 170000
_NTILES = 32
_EPT = 5344
_NB = _EPT // 16
_E_PAD = _NTILES * _EPT
_BN = 1000


def _splat(v, j):
    dn = lax.GatherDimensionNumbers(
        offset_dims=(), collapsed_slice_dims=(0,), start_index_map=(0,))
    return lax.gather(v, jnp.full((16, 1), j, jnp.int32), dn,
                      slice_sizes=(1,),
                      mode=lax.GatherScatterMode.PROMISE_IN_BOUNDS)


def _mm_body(x_ref, w_ref, s_ref, d_ref, h_ref, as_ref, ad_ref):
    c = pl.program_id(1)
    h = jnp.dot(x_ref[...], w_ref[...], preferred_element_type=jnp.float32)
    h_ref[0] = h
    sv = jnp.dot(h, s_ref[...], preferred_element_type=jnp.float32)
    dv = jnp.dot(h, d_ref[...], preferred_element_type=jnp.float32)

    @pl.when(c == 0)
    def _():
        as_ref[...] = sv
        ad_ref[...] = dv

    @pl.when(c > 0)
    def _():
        as_ref[...] = as_ref[...] + sv
        ad_ref[...] = ad_ref[...] + dv


def _mm(x, w, att_s, att_d, c_tot, H):
    k = x.shape[1]
    return pl.pallas_call(
        _mm_body,
        grid=(_N // _BN, c_tot),
        in_specs=[
            pl.BlockSpec((_BN, k), lambda n, c: (n, 0)),
            pl.BlockSpec((k, 128), lambda n, c: (0, c)),
            pl.BlockSpec((128, H), lambda n, c: (c, 0)),
            pl.BlockSpec((128, H), lambda n, c: (c, 0)),
        ],
        out_specs=[
            pl.BlockSpec((1, _BN, 128), lambda n, c: (c, n, 0)),
            pl.BlockSpec((_BN, H), lambda n, c: (n, 0)),
            pl.BlockSpec((_BN, H), lambda n, c: (n, 0)),
        ],
        out_shape=[
            jax.ShapeDtypeStruct((c_tot, _N, 128), jnp.float32),
            jax.ShapeDtypeStruct((_N, H), jnp.float32),
            jax.ShapeDtypeStruct((_N, H), jnp.float32),
        ],
    )(x, w, att_s, att_d)


def _make_edge_a(H):
    mesh = plsc.VectorSubcoreMesh(core_axis_name="c", subcore_axis_name="s")

    @functools.partial(
        pl.kernel,
        out_type=jax.ShapeDtypeStruct((_NTILES * H * _NP,), jnp.float32),
        mesh=mesh,
        compiler_params=pltpu.CompilerParams(needs_layout_passes=False),
        scratch_types=[
            pltpu.VMEM((_EPT,), jnp.int32),
            pltpu.VMEM((_EPT,), jnp.int32),
            pltpu.VMEM((_N,), jnp.float32),
            pltpu.VMEM((_N,), jnp.float32),
            pltpu.VMEM((_NP,), jnp.float32),
        ],
    )
    def edge_a(asrc, adst, src, dst, den_out,
               src_v, dst_v, as_v, ad_v, den_v):
        core = lax.axis_index("c")
        sub = lax.axis_index("s")
        wid = core * 16 + sub
        base = pl.multiple_of(wid * _EPT, 8)
        pltpu.sync_copy(src.at[pl.ds(base, _EPT)], src_v)
        pltpu.sync_copy(dst.at[pl.ds(base, _EPT)], dst_v)

        for h in range(H):
            pltpu.sync_copy(asrc.at[pl.ds(h * _N, _N)], as_v)
            pltpu.sync_copy(adst.at[pl.ds(h * _N, _N)], ad_v)

            def _zl(i, _):
                den_v[pl.ds(i * 16, 16)] = jnp.zeros((16,), jnp.float32)
                return 0

            lax.fori_loop(0, _NP // 16, _zl, 0)

            def _el(b, _):
                sl16 = pl.ds(b * 16, 16)
                s16 = src_v[sl16]
                d16 = dst_v[sl16]
                a = plsc.load_gather(as_v, [s16]) + plsc.load_gather(ad_v, [d16])
                a = jnp.where(a >= 0, a, a * jnp.float32(0.2))
                v = jnp.exp(a)
                gi = base + b * 16 + lax.iota(jnp.int32, 16)
                v = jnp.where(gi < _E_TRUE, v, jnp.float32(0.0))
                plsc.addupdate_scatter(den_v, [d16], v)
                return 0

            lax.fori_loop(0, _NB, _el, 0)
            doff = pl.multiple_of((wid * H + h) * _NP, 8)
            pltpu.sync_copy(den_v, den_out.at[pl.ds(doff, _NP)])

    return edge_a


def _inv_body(dp_ref, inv_ref):
    inv_ref[...] = 1.0 / (jnp.sum(dp_ref[...], axis=0) + jnp.float32(1e-16))


def _inv(den_parts, H):
    return pl.pallas_call(
        _inv_body,
        out_shape=jax.ShapeDtypeStruct((H, _NP), jnp.float32),
    )(den_parts)


def _make_edge_b(H, halves, c_feat):
    mesh = plsc.VectorSubcoreMesh(core_axis_name="c", subcore_axis_name="s")

    @functools.partial(
        pl.kernel,
        out_type=jax.ShapeDtypeStruct((2, c_feat, _NP, 128), jnp.float32),
        mesh=mesh,
        compiler_params=pltpu.CompilerParams(needs_layout_passes=False),
        scratch_types=[
            pltpu.VMEM((_EPT,), jnp.int32),
            pltpu.VMEM((_EPT,), jnp.int32),
            pltpu.VMEM((_EPT,), jnp.float32),
            pltpu.VMEM((_N,), jnp.float32),
            pltpu.VMEM((_N,), jnp.float32),
            pltpu.VMEM((2, 16, 128), jnp.float32),
            pltpu.VMEM((2, 16, 128), jnp.float32),
            pltpu.VMEM((2, 16), jnp.int32),
            pltpu.VMEM((2, 16), jnp.int32),
            pltpu.VMEM((640,), jnp.float32),
            pltpu.VMEM_SHARED((_NP, 128), jnp.float32),
            pltpu.SemaphoreType.DMA,
            pltpu.SemaphoreType.DMA,
            pltpu.SemaphoreType.DMA,
            pltpu.SemaphoreType.DMA,
        ],
    )
    def edge_b(asrc, adst, src, dst, hflat, inv_hbm, zrow, num_out,
               src_v, dst_v, val_v, as_v, ad_v, rows_v, srows_v, sidx_v,
               didx_v, invs_v, num_sh, sem0, sem1, sem2, sem3):
        gsems = (sem0, sem1)
        ssems = (sem2, sem3)
        core = lax.axis_index("c")
        sub = lax.axis_index("s")
        wid = core * 16 + sub
        base = pl.multiple_of(wid * _EPT, 8)
        sbase = pl.multiple_of(sub * 640, 8)
        pltpu.sync_copy(src.at[pl.ds(base, _EPT)], src_v)
        pltpu.sync_copy(dst.at[pl.ds(base, _EPT)], dst_v)

        def _chunk(c, _):
            hh = c // halves
            aoff = pl.multiple_of(hh * _N, 8)
            pltpu.sync_copy(asrc.at[pl.ds(aoff, _N)], as_v)
            pltpu.sync_copy(adst.at[pl.ds(aoff, _N)], ad_v)

            def _el(b, _):
                sl16 = pl.ds(b * 16, 16)
                a = (plsc.load_gather(as_v, [src_v[sl16]])
                     + plsc.load_gather(ad_v, [dst_v[sl16]]))
                a = jnp.where(a >= 0, a, a * jnp.float32(0.2))
                v = jnp.exp(a)
                gi = base + b * 16 + lax.iota(jnp.int32, 16)
                v = jnp.where(gi < _E_TRUE, v, jnp.float32(0.0))
                val_v[sl16] = v
                return 0

            lax.fori_loop(0, _NB, _el, 0)
            def _zz(r, _):
                pltpu.sync_copy(zrow, num_sh.at[pl.ds(sbase + r * 128, 128)])
                return 0

            lax.fori_loop(0, 5, _zz, 0)
            plsc.subcore_barrier()
            for slot in range(2):
                sidx_v[slot] = src_v[pl.ds(slot * 16, 16)] + c * _N
                pltpu.async_copy(hflat.at[sidx_v.at[slot]], rows_v.at[slot],
                                 gsems[slot])

            def _pb(i, _):
                for slot in range(2):
                    b = i * 2 + slot
                    pltpu.make_async_copy(
                        hflat.at[sidx_v.at[slot]], rows_v.at[slot],
                        gsems[slot]).wait()

                    @pl.when(i > 0)
                    def _():
                        pltpu.make_async_copy(
                            srows_v.at[slot], num_sh.at[didx_v.at[slot]],
                            ssems[slot]).wait()
                    didx_v[slot] = dst_v[pl.ds(b * 16, 16)]
                    v16 = val_v[pl.ds(b * 16, 16)]
                    for j in range(16):
                        sp = _splat(v16, j)
                        for g in range(8):
                            sl16 = pl.ds(g * 16, 16)
                            srows_v[slot, j, sl16] = rows_v[slot, j, sl16] * sp
                    bnext = jnp.minimum(b + 2, _NB - 1)
                    sidx_v[slot] = src_v[pl.ds(bnext * 16, 16)] + c * _N
                    pltpu.async_copy(hflat.at[sidx_v.at[slot]], rows_v.at[slot],
                                     gsems[slot])
                    pltpu.async_copy(srows_v.at[slot], num_sh.at[didx_v.at[slot]],
                                     ssems[slot], add=True)
                return 0

            lax.fori_loop(0, _NB // 2, _pb, 0)
            for slot in range(2):
                pltpu.make_async_copy(
                    hflat.at[sidx_v.at[slot]], rows_v.at[slot], gsems[slot]).wait()
                pltpu.make_async_copy(
                    srows_v.at[slot], num_sh.at[didx_v.at[slot]],
                    ssems[slot]).wait()
            plsc.subcore_barrier()
            ioff = pl.multiple_of(hh * _NP + sbase, 8)
            pltpu.sync_copy(inv_hbm.at[pl.ds(ioff, 640)], invs_v)

            def _wr(r, _):
                wb = rows_v.at[0]
                pltpu.sync_copy(num_sh.at[pl.ds(sbase + r * 16, 16)], wb)
                iv16 = invs_v[pl.ds(r * 16, 16)]
                for j in range(16):
                    sp = _splat(iv16, j)
                    for g in range(8):
                        sl16 = pl.ds(g * 16, 16)
                        rows_v[0, j, sl16] = rows_v[0, j, sl16] * sp
                pltpu.sync_copy(wb, num_out.at[core, c, pl.ds(sbase + r * 16, 16)])
                return 0

            lax.fori_loop(0, 40, _wr, 0)
            return 0

        lax.fori_loop(0, c_feat, _chunk, 0)

    return edge_b


_edge_a4 = _make_edge_a(H=4)
_edge_a6 = _make_edge_a(H=6)
_edge_b12 = _make_edge_b(H=4, halves=2, c_feat=8)
_edge_b3 = _make_edge_b(H=6, halves=1, c_feat=6)


def _c12_body(num_ref, hc_ref, bg_ref, bl_ref, out_ref):
    c = pl.program_id(0)
    v = num_ref[0, 0] + num_ref[1, 0]
    v = v + bg_ref[c][None, :] + hc_ref[0] + bl_ref[c][None, :]
    out_ref[...] = jnp.where(v > 0, v, jnp.exp(jnp.minimum(v, 0.0)) - 1.0)


def _c12(num, hc, bg, bl, c_feat):
    return pl.pallas_call(
        _c12_body,
        grid=(c_feat, _N // _BN),
        in_specs=[
            pl.BlockSpec((2, 1, _BN, 128), lambda c, n: (0, c, n, 0)),
            pl.BlockSpec((1, _BN, 128), lambda c, n: (c_feat + c, n, 0)),
            pl.BlockSpec((c_feat, 128), lambda c, n: (0, 0)),
            pl.BlockSpec((c_feat, 128), lambda c, n: (0, 0)),
        ],
        out_specs=pl.BlockSpec((_BN, 128), lambda c, n: (n, c)),
        out_shape=jax.ShapeDtypeStruct((_N, c_feat * 128), jnp.float32),
    )(num, hc, bg, bl)


def _c3_body(num_ref, hc_ref, b3_ref, bl3_ref, out_ref):
    acc = num_ref[0, 0] + num_ref[1, 0]
    for h in range(1, 6):
        acc = acc + num_ref[0, h] + num_ref[1, h]
    out_ref[...] = (acc * jnp.float32(1.0 / 6.0)
                    + b3_ref[0][None, :] + hc_ref[0] + bl3_ref[0][None, :])


def _c3(num, hc, b3, bl3):
    return pl.pallas_call(
        _c3_body,
        grid=(_N // _BN,),
        in_specs=[
            pl.BlockSpec((2, 6, _BN, 128), lambda n: (0, 0, n, 0)),
            pl.BlockSpec((1, _BN, 128), lambda n: (6, n, 0)),
            pl.BlockSpec((1, 128), lambda n: (0, 0)),
            pl.BlockSpec((1, 128), lambda n: (0, 0)),
        ],
        out_specs=pl.BlockSpec((_BN, 128), lambda n: (n, 0)),
        out_shape=jax.ShapeDtypeStruct((_N, 128), jnp.float32),
    )(num, hc, b3, bl3)


def _att_mats(att, c_tot, H, halves):
    rows = att.reshape(H * halves, 128)
    rows = jnp.concatenate(
        [rows, jnp.zeros((c_tot - H * halves, 128), jnp.float32)], axis=0)
    eye = jnp.repeat(jnp.eye(H, dtype=jnp.float32), halves, axis=0)
    oh = jnp.concatenate([eye, jnp.zeros((c_tot - H * halves, H))], axis=0)
    return (rows[:, :, None] * oh[:, None, :]).reshape(c_tot * 128, H)


def _gat_layer(x, W, att_s, att_d, Wl, edge_a, edge_b, src, dst, zrow,
               c_tot, c_feat, H, halves):
    wc = jnp.concatenate([W, Wl], axis=1)
    a_s = _att_mats(att_s, c_tot, H, halves)
    a_d = _att_mats(att_d, c_tot, H, halves)
    hc, s, d = _mm(x, wc, a_s, a_d, c_tot, H)
    st, dt = s.T.reshape(-1), d.T.reshape(-1)
    den = edge_a(st, dt, src, dst)
    inv = _inv(den.reshape(_NTILES, H, _NP), H)
    num = edge_b(st, dt, src, dst, hc.reshape(c_tot * _N, 128),
                 inv.reshape(-1), zrow)
    return hc, num


def kernel(x, edge_index, W1, as1, ad1, b1, Wl1, bl1, W2, as2, ad2, b2,
           Wl2, bl2, W3, as3, ad3, b3, Wl3, bl3):
    ar = jnp.arange(_N, dtype=jnp.int32)
    src = jnp.concatenate([edge_index[0].astype(jnp.int32), ar])
    dst = jnp.concatenate([edge_index[1].astype(jnp.int32), ar])
    pad = _E_PAD - _E_TRUE
    src = jnp.pad(src, (0, pad))
    dst = jnp.pad(dst, (0, pad))
    zrow = jnp.zeros((128, 128), jnp.float32)

    hc, num = _gat_layer(x, W1, as1, ad1, Wl1, _edge_a4, _edge_b12,
                         src, dst, zrow, c_tot=16, c_feat=8, H=4, halves=2)
    h1 = _c12(num, hc, b1.reshape(8, 128), bl1.reshape(8, 128), c_feat=8)

    hc, num = _gat_layer(h1, W2, as2, ad2, Wl2, _edge_a4, _edge_b12,
                         src, dst, zrow, c_tot=16, c_feat=8, H=4, halves=2)
    h2 = _c12(num, hc, b2.reshape(8, 128), bl2.reshape(8, 128), c_feat=8)

    hc, num = _gat_layer(h2, W3, as3, ad3, Wl3, _edge_a6, _edge_b3,
                         src, dst, zrow, c_tot=7, c_feat=6, H=6, halves=1)
    return _c3(num, hc, b3.reshape(1, 128), bl3.reshape(1, 128))
